# Initial kernel scaffold; baseline (speedup 1.0000x reference)
import jax
import jax.numpy as jnp
from jax.experimental import pallas as pl


def kernel(data, edge_index_0, edge_type_0, node_type_0, edge_index_1, edge_type_1, node_type_1, depth, W_down0, b_down0, gns_down0, gnb_down0, W_conv0, gns_conv0, gnb_conv0, W_down1, b_down1, gns_down1, gnb_down1, W_conv1, gns_conv1, gnb_conv1):
    raise NotImplementedError("write your pallas kernel here")


# speedup vs baseline: 15.5528x; 15.5528x over previous
import functools

import jax
import jax.numpy as jnp
from jax import lax
from jax.experimental import pallas as pl
from jax.experimental.pallas import tpu as pltpu
from jax.experimental.pallas import tpu_sc as plsc

NC, NS, LANES = 2, 16, 16
NW = NC * NS
GROUPS = 8
EPS = 1e-5
NTYPES = 7


def _gn(h, gns, gnb, C):
    g = C // GROUPS
    r = lax.broadcasted_iota(jnp.int32, (C, C), 0) // g
    c = lax.broadcasted_iota(jnp.int32, (C, C), 1) // g
    mg = (r == c).astype(jnp.float32) / g
    m = jnp.dot(h, mg, preferred_element_type=jnp.float32)
    e2 = jnp.dot(h * h, mg, preferred_element_type=jnp.float32)
    v = e2 - m * m
    return (h - m) * lax.rsqrt(v + EPS) * gns + gnb


def _tc_stage_body(nt_ref, d2_ref, w8_ref, b8_ref, gns_ref, gnb_ref,
                   wx_ref, wo_ref, bcat_ref, out_ref, *, C, NT):
    h = jnp.dot(d2_ref[:], w8_ref[:], preferred_element_type=jnp.float32)
    h = h + b8_ref[:]
    x = jax.nn.gelu(_gn(h, gns_ref[:], gnb_ref[:], C))
    oh = (nt_ref[:] == lax.broadcasted_iota(jnp.int32, (1, NT), 1))
    oh = oh.astype(jnp.float32)
    out_ref[:] = (jnp.dot(x, wx_ref[:], preferred_element_type=jnp.float32)
                  + jnp.dot(oh, wo_ref[:], preferred_element_type=jnp.float32)
                  + bcat_ref[:])


def _tc_stage(nt2, d2, w8, b8, gns, gnb, wx, wo, bcat, *, C, NT, bn):
    N, K = d2.shape
    TW = wx.shape[1]
    return pl.pallas_call(
        functools.partial(_tc_stage_body, C=C, NT=NT),
        grid=(N // bn,),
        in_specs=[
            pl.BlockSpec((bn, 1), lambda i: (i, 0)),
            pl.BlockSpec((bn, K), lambda i: (i, 0)),
            pl.BlockSpec((K, C), lambda i: (0, 0)),
            pl.BlockSpec((1, C), lambda i: (0, 0)),
            pl.BlockSpec((1, C), lambda i: (0, 0)),
            pl.BlockSpec((1, C), lambda i: (0, 0)),
            pl.BlockSpec((C, TW), lambda i: (0, 0)),
            pl.BlockSpec((NT, TW), lambda i: (0, 0)),
            pl.BlockSpec((1, TW), lambda i: (0, 0)),
        ],
        out_specs=pl.BlockSpec((bn, TW), lambda i: (i, 0)),
        out_shape=jax.ShapeDtypeStruct((N, TW), jnp.float32),
    )(nt2, d2, w8, b8, gns, gnb, wx, wo, bcat)


def _tc_combine_body(p0_ref, p1_ref, gns_ref, gnb_ref, out_ref, *, C):
    s = p0_ref[:] + p1_ref[:]
    deg = jnp.maximum(s[:, C:C + 1], 1.0)
    agg = s[:, :C] / deg
    out_ref[:] = jax.nn.gelu(_gn(agg, gns_ref[:], gnb_ref[:], C))


def _tc_combine(p0, p1, gns, gnb, *, C, bn):
    N, W = p0.shape
    return pl.pallas_call(
        functools.partial(_tc_combine_body, C=C),
        grid=(N // bn,),
        in_specs=[
            pl.BlockSpec((bn, W), lambda i: (i, 0)),
            pl.BlockSpec((bn, W), lambda i: (i, 0)),
            pl.BlockSpec((1, C), lambda i: (0, 0)),
            pl.BlockSpec((1, C), lambda i: (0, 0)),
        ],
        out_specs=pl.BlockSpec((bn, C), lambda i: (i, 0)),
        out_shape=jax.ShapeDtypeStruct((N, C), jnp.float32),
    )(p0, p1, gns, gnb)


def _sc_conv(xt_flat, src, typ, dst, *, NACC, WIDTH, B=128):
    E = src.shape[0]
    ew = E // NW
    nchunk = ew // B
    rps = NACC // NS
    mesh = plsc.VectorSubcoreMesh(core_axis_name="c", subcore_axis_name="s",
                                  num_cores=NC, num_subcores=NS)

    @functools.partial(
        pl.kernel,
        out_type=jax.ShapeDtypeStruct((NC, NACC, WIDTH), jnp.float32),
        mesh=mesh,
        compiler_params=pltpu.CompilerParams(use_tc_tiling_on_sc=False),
        scratch_types=[
            pltpu.VMEM((B,), jnp.int32),
            pltpu.VMEM((B,), jnp.int32),
            pltpu.VMEM((1, B), jnp.int32),
            pltpu.VMEM((B,), jnp.int32),
            pltpu.VMEM((B, WIDTH), jnp.float32),
            pltpu.VMEM_SHARED((NACC, WIDTH), jnp.float32),
            pltpu.SemaphoreType.DMA,
        ],
    )
    def k(xt_hbm, src_hbm, typ_hbm, dst_hbm, out_hbm,
          srcb, typb, dstb, idxb, rowsb, acc, sem):
        cid = lax.axis_index("c")
        sid = lax.axis_index("s")
        wid = sid * NC + cid

        @pl.loop(0, B)
        def _(i):
            for cc in range(WIDTH // LANES):
                rowsb[i, pl.ds(cc * LANES, LANES)] = jnp.zeros(
                    (LANES,), jnp.float32)
        rbase = sid * rps
        for kk in range(rps // B):
            pltpu.sync_copy(rowsb, acc.at[pl.ds(rbase + kk * B, B)])
        plsc.subcore_barrier()

        base = wid * ew

        @pl.loop(0, nchunk)
        def _(c):
            off = base + c * B
            pltpu.sync_copy(src_hbm.at[pl.ds(off, B)], srcb)
            pltpu.sync_copy(typ_hbm.at[pl.ds(off, B)], typb)
            pltpu.sync_copy(dst_hbm.at[pl.ds(off, B)], dstb.at[0])
            for j in range(B // LANES):
                sl = pl.ds(j * LANES, LANES)
                idxb[sl] = srcb[sl] * NTYPES + typb[sl]
            pltpu.async_copy(xt_hbm.at[idxb], rowsb, sem).wait()
            pltpu.sync_copy(rowsb, acc.at[dstb.at[0]], add=True)

        plsc.subcore_barrier()
        for kk in range(rps // B):
            r0 = rbase + kk * B
            pltpu.sync_copy(acc.at[pl.ds(r0, B)], rowsb)
            pltpu.sync_copy(rowsb, out_hbm.at[cid, pl.ds(r0, B)])

    return k(xt_flat, src, typ, dst)


def _prep_conv_weights(W_conv, C, NT, WPAD):
    Cout = W_conv.shape[2]
    wx = jnp.transpose(W_conv[:, :C, :], (1, 0, 2))
    wx = jnp.pad(wx, ((0, 0), (0, 0), (0, WPAD - Cout)))
    wx = wx.reshape(C, NTYPES * WPAD)
    wo = jnp.transpose(W_conv[:, C:, :], (1, 0, 2))
    wo = jnp.pad(wo, ((0, 0), (0, 0), (0, WPAD - Cout)))
    wo = wo.reshape(NT, NTYPES * WPAD)
    bcat = jnp.zeros((NTYPES, WPAD), jnp.float32).at[:, Cout].set(1.0)
    bcat = bcat.reshape(1, NTYPES * WPAD)
    return wx, wo, bcat


def kernel(data, edge_index_0, edge_type_0, node_type_0,
           edge_index_1, edge_type_1, node_type_1, depth,
           W_down0, b_down0, gns_down0, gnb_down0,
           W_conv0, gns_conv0, gnb_conv0,
           W_down1, b_down1, gns_down1, gnb_down1,
           W_conv1, gns_conv1, gnb_conv1):
    del depth
    N0, C0 = data.shape
    N1 = node_type_0.shape[0]
    N2 = node_type_1.shape[0]
    C1 = W_down1.shape[1]
    NT0 = W_conv0.shape[1] - C0
    NT1 = W_conv1.shape[1] - C1
    W0, W1 = 48, 80

    d2 = data.reshape(N1, 8 * C0)
    w8_0 = jnp.tile(W_down0 / 8.0, (8, 1))
    wx0, wo0, bcat0 = _prep_conv_weights(W_conv0, C0, NT0, W0)
    xt0 = _tc_stage(node_type_0.reshape(N1, 1), d2, w8_0,
                    b_down0.reshape(1, C0), gns_down0.reshape(1, C0),
                    gnb_down0.reshape(1, C0), wx0, wo0, bcat0,
                    C=C0, NT=NT0, bn=2048)
    part0 = _sc_conv(xt0.reshape(N1 * NTYPES, W0),
                     edge_index_0[0], edge_type_0, edge_index_0[1],
                     NACC=N1, WIDTH=W0)
    x1 = _tc_combine(part0[0], part0[1], gns_conv0.reshape(1, C0),
                     gnb_conv0.reshape(1, C0), C=C0, bn=4096)

    x1p = x1.reshape(N2, 8 * C0)
    w8_1 = jnp.tile(W_down1 / 8.0, (8, 1))
    wx1, wo1, bcat1 = _prep_conv_weights(W_conv1, C1, NT1, W1)
    xt1 = _tc_stage(node_type_1.reshape(N2, 1), x1p, w8_1,
                    b_down1.reshape(1, C1), gns_down1.reshape(1, C1),
                    gnb_down1.reshape(1, C1), wx1, wo1, bcat1,
                    C=C1, NT=NT1, bn=2048)
    part1 = _sc_conv(xt1.reshape(N2 * NTYPES, W1),
                     edge_index_1[0], edge_type_1, edge_index_1[1],
                     NACC=N2, WIDTH=W1)
    out = _tc_combine(part1[0], part1[1], gns_conv1.reshape(1, C1),
                      gnb_conv1.reshape(1, C1), C=C1, bn=2048)
    return out


# speedup vs baseline: 24.4338x; 1.5710x over previous
import functools

import jax
import jax.numpy as jnp
from jax import lax
from jax.experimental import pallas as pl
from jax.experimental.pallas import tpu as pltpu
from jax.experimental.pallas import tpu_sc as plsc

NC, NS, LANES = 2, 16, 16
NW = NC * NS
GROUPS = 8
EPS = 1e-5
NTYPES = 7


def _gn(h, gns, gnb, C):
    g = C // GROUPS
    r = lax.broadcasted_iota(jnp.int32, (C, C), 0) // g
    c = lax.broadcasted_iota(jnp.int32, (C, C), 1) // g
    mg = (r == c).astype(jnp.float32) / g
    m = jnp.dot(h, mg, preferred_element_type=jnp.float32)
    e2 = jnp.dot(h * h, mg, preferred_element_type=jnp.float32)
    v = e2 - m * m
    return (h - m) * lax.rsqrt(v + EPS) * gns + gnb


def _tc_stage_body(nt_ref, d2_ref, w8_ref, b8_ref, gns_ref, gnb_ref,
                   wx_ref, wo_ref, bcat_ref, out_ref, *, C, NT):
    h = jnp.dot(d2_ref[:], w8_ref[:], preferred_element_type=jnp.float32)
    h = h + b8_ref[:]
    x = jax.nn.gelu(_gn(h, gns_ref[:], gnb_ref[:], C))
    oh = (nt_ref[:] == lax.broadcasted_iota(jnp.int32, (1, NT), 1))
    oh = oh.astype(jnp.float32)
    out_ref[:] = (jnp.dot(x, wx_ref[:], preferred_element_type=jnp.float32)
                  + jnp.dot(oh, wo_ref[:], preferred_element_type=jnp.float32)
                  + bcat_ref[:])


def _tc_stage(nt2, d2, w8, b8, gns, gnb, wx, wo, bcat, *, C, NT, bn):
    N, K = d2.shape
    TW = wx.shape[1]
    return pl.pallas_call(
        functools.partial(_tc_stage_body, C=C, NT=NT),
        grid=(N // bn,),
        in_specs=[
            pl.BlockSpec((bn, 1), lambda i: (i, 0)),
            pl.BlockSpec((bn, K), lambda i: (i, 0)),
            pl.BlockSpec((K, C), lambda i: (0, 0)),
            pl.BlockSpec((1, C), lambda i: (0, 0)),
            pl.BlockSpec((1, C), lambda i: (0, 0)),
            pl.BlockSpec((1, C), lambda i: (0, 0)),
            pl.BlockSpec((C, TW), lambda i: (0, 0)),
            pl.BlockSpec((NT, TW), lambda i: (0, 0)),
            pl.BlockSpec((1, TW), lambda i: (0, 0)),
        ],
        out_specs=pl.BlockSpec((bn, TW), lambda i: (i, 0)),
        out_shape=jax.ShapeDtypeStruct((N, TW), jnp.float32),
    )(nt2, d2, w8, b8, gns, gnb, wx, wo, bcat)


def _tc_combine_body(p0_ref, p1_ref, gns_ref, gnb_ref, out_ref, *, C):
    s = p0_ref[:] + p1_ref[:]
    deg = jnp.maximum(s[:, C:C + 1], 1.0)
    agg = s[:, :C] / deg
    out_ref[:] = jax.nn.gelu(_gn(agg, gns_ref[:], gnb_ref[:], C))


def _tc_combine(p0, p1, gns, gnb, *, C, bn):
    N, W = p0.shape
    return pl.pallas_call(
        functools.partial(_tc_combine_body, C=C),
        grid=(N // bn,),
        in_specs=[
            pl.BlockSpec((bn, W), lambda i: (i, 0)),
            pl.BlockSpec((bn, W), lambda i: (i, 0)),
            pl.BlockSpec((1, C), lambda i: (0, 0)),
            pl.BlockSpec((1, C), lambda i: (0, 0)),
        ],
        out_specs=pl.BlockSpec((bn, C), lambda i: (i, 0)),
        out_shape=jax.ShapeDtypeStruct((N, C), jnp.float32),
    )(p0, p1, gns, gnb)


def _sc_conv(xt_flat, src, typ, dst2d, *, NACC, WIDTH, B=128, NBUF=8):
    E = src.shape[0]
    ew = E // NW
    nchunk = ew // B
    nouter = nchunk // NBUF
    rps = NACC // NS
    mesh = plsc.VectorSubcoreMesh(core_axis_name="c", subcore_axis_name="s",
                                  num_cores=NC, num_subcores=NS)

    @functools.partial(
        pl.kernel,
        out_type=jax.ShapeDtypeStruct((NC, NACC, WIDTH), jnp.float32),
        mesh=mesh,
        compiler_params=pltpu.CompilerParams(use_tc_tiling_on_sc=False),
        scratch_types=[
            pltpu.VMEM((ew,), jnp.int32),
            pltpu.VMEM((ew,), jnp.int32),
            pltpu.VMEM((nchunk, B), jnp.int32),
            pltpu.VMEM((ew,), jnp.int32),
            pltpu.VMEM((NBUF, B, WIDTH), jnp.float32),
            pltpu.VMEM_SHARED((NACC, WIDTH), jnp.float32),
            [pltpu.SemaphoreType.DMA] * NBUF,
            [pltpu.SemaphoreType.DMA] * NBUF,
        ],
    )
    def k(xt_hbm, src_hbm, typ_hbm, dst_hbm, out_hbm,
          srcb, typb, dstb, idxb, rowsb, acc, gsem, ssem):
        cid = lax.axis_index("c")
        sid = lax.axis_index("s")
        wid = sid * NC + cid
        base = wid * ew

        pltpu.sync_copy(src_hbm.at[pl.ds(base, ew)], srcb)
        pltpu.sync_copy(typ_hbm.at[pl.ds(base, ew)], typb)
        pltpu.sync_copy(dst_hbm.at[pl.ds(wid * nchunk, nchunk)], dstb)

        @pl.loop(0, B)
        def _(i):
            for cc in range(WIDTH // LANES):
                rowsb[0, i, pl.ds(cc * LANES, LANES)] = jnp.zeros(
                    (LANES,), jnp.float32)
        rbase = sid * rps
        for kk in range(rps // B):
            pltpu.sync_copy(rowsb.at[0], acc.at[pl.ds(rbase + kk * B, B)])

        @pl.loop(0, ew // LANES)
        def _(j):
            sl = pl.ds(j * LANES, LANES)
            idxb[sl] = srcb[sl] * NTYPES + typb[sl]
        plsc.subcore_barrier()

        def fire_gather(c, b):
            pltpu.async_copy(
                xt_hbm.at[idxb.at[pl.ds(c * B, B)]], rowsb.at[b], gsem[b])

        def wait_gather(b):
            pltpu.make_async_copy(
                xt_hbm.at[idxb.at[pl.ds(0, B)]], rowsb.at[b], gsem[b]).wait()

        def fire_scatter(c, b):
            pltpu.async_copy(
                rowsb.at[b], acc.at[dstb.at[c]], ssem[b], add=True)

        def wait_scatter(b):
            pltpu.make_async_copy(
                rowsb.at[b], acc.at[dstb.at[0]], ssem[b]).wait()

        for b in range(NBUF):
            fire_gather(b, b)

        @pl.loop(0, nouter - 1)
        def _(i):
            c0 = i * NBUF
            for b in range(NBUF):
                wait_gather(b)
                fire_scatter(c0 + b, b)
            for b in range(NBUF):
                wait_scatter(b)
                fire_gather(c0 + NBUF + b, b)

        c0 = (nouter - 1) * NBUF
        for b in range(NBUF):
            wait_gather(b)
            fire_scatter(c0 + b, b)
        for b in range(NBUF):
            wait_scatter(b)

        plsc.subcore_barrier()
        for kk in range(rps // B):
            r0 = rbase + kk * B
            pltpu.sync_copy(acc.at[pl.ds(r0, B)], rowsb.at[0])
            pltpu.sync_copy(rowsb.at[0], out_hbm.at[cid, pl.ds(r0, B)])

    return k(xt_flat, src, typ, dst2d)


def _prep_conv_weights(W_conv, C, NT, WPAD):
    Cout = W_conv.shape[2]
    wx = jnp.transpose(W_conv[:, :C, :], (1, 0, 2))
    wx = jnp.pad(wx, ((0, 0), (0, 0), (0, WPAD - Cout)))
    wx = wx.reshape(C, NTYPES * WPAD)
    wo = jnp.transpose(W_conv[:, C:, :], (1, 0, 2))
    wo = jnp.pad(wo, ((0, 0), (0, 0), (0, WPAD - Cout)))
    wo = wo.reshape(NT, NTYPES * WPAD)
    bcat = jnp.zeros((NTYPES, WPAD), jnp.float32).at[:, Cout].set(1.0)
    bcat = bcat.reshape(1, NTYPES * WPAD)
    return wx, wo, bcat


def kernel(data, edge_index_0, edge_type_0, node_type_0,
           edge_index_1, edge_type_1, node_type_1, depth,
           W_down0, b_down0, gns_down0, gnb_down0,
           W_conv0, gns_conv0, gnb_conv0,
           W_down1, b_down1, gns_down1, gnb_down1,
           W_conv1, gns_conv1, gnb_conv1):
    del depth
    N0, C0 = data.shape
    N1 = node_type_0.shape[0]
    N2 = node_type_1.shape[0]
    C1 = W_down1.shape[1]
    NT0 = W_conv0.shape[1] - C0
    NT1 = W_conv1.shape[1] - C1
    W0, W1 = 48, 80

    d2 = data.reshape(N1, 8 * C0)
    w8_0 = jnp.tile(W_down0 / 8.0, (8, 1))
    wx0, wo0, bcat0 = _prep_conv_weights(W_conv0, C0, NT0, W0)
    xt0 = _tc_stage(node_type_0.reshape(N1, 1), d2, w8_0,
                    b_down0.reshape(1, C0), gns_down0.reshape(1, C0),
                    gnb_down0.reshape(1, C0), wx0, wo0, bcat0,
                    C=C0, NT=NT0, bn=2048)
    part0 = _sc_conv(xt0.reshape(N1 * NTYPES, W0),
                     edge_index_0[0], edge_type_0,
                     edge_index_0[1].reshape(-1, 128),
                     NACC=N1, WIDTH=W0)
    x1 = _tc_combine(part0[0], part0[1], gns_conv0.reshape(1, C0),
                     gnb_conv0.reshape(1, C0), C=C0, bn=4096)

    x1p = x1.reshape(N2, 8 * C0)
    w8_1 = jnp.tile(W_down1 / 8.0, (8, 1))
    wx1, wo1, bcat1 = _prep_conv_weights(W_conv1, C1, NT1, W1)
    xt1 = _tc_stage(node_type_1.reshape(N2, 1), x1p, w8_1,
                    b_down1.reshape(1, C1), gns_down1.reshape(1, C1),
                    gnb_down1.reshape(1, C1), wx1, wo1, bcat1,
                    C=C1, NT=NT1, bn=2048)
    part1 = _sc_conv(xt1.reshape(N2 * NTYPES, W1),
                     edge_index_1[0], edge_type_1,
                     edge_index_1[1].reshape(-1, 128),
                     NACC=N2, WIDTH=W1)
    out = _tc_combine(part1[0], part1[1], gns_conv1.reshape(1, C1),
                      gnb_conv1.reshape(1, C1), C=C1, bn=2048)
    return out


# speedup vs baseline: 25.6845x; 1.0512x over previous
import functools

import jax
import jax.numpy as jnp
from jax import lax
from jax.experimental import pallas as pl
from jax.experimental.pallas import tpu as pltpu
from jax.experimental.pallas import tpu_sc as plsc

NC, NS, LANES = 2, 16, 16
NW = NC * NS
GROUPS = 8
EPS = 1e-5
NTYPES = 7


def _gn(h, gns, gnb, C):
    g = C // GROUPS
    r = lax.broadcasted_iota(jnp.int32, (C, C), 0) // g
    c = lax.broadcasted_iota(jnp.int32, (C, C), 1) // g
    mg = (r == c).astype(jnp.float32) / g
    m = jnp.dot(h, mg, preferred_element_type=jnp.float32)
    e2 = jnp.dot(h * h, mg, preferred_element_type=jnp.float32)
    v = e2 - m * m
    return (h - m) * lax.rsqrt(v + EPS) * gns + gnb


def _down(xin, w_ref, b_ref, gns_ref, gnb_ref, C):
    n8, cin = xin.shape
    xp = jnp.mean(xin.reshape(n8 // 8, 8, cin), axis=1)
    h = jnp.dot(xp, w_ref[:], preferred_element_type=jnp.float32) + b_ref[:]
    return jax.nn.gelu(_gn(h, gns_ref[:], gnb_ref[:], C))


def _table(x, nt_ref, wx_ref, wo_ref, bcat_ref, NT):
    oh = (nt_ref[:] == lax.broadcasted_iota(jnp.int32, (1, NT), 1))
    oh = oh.astype(jnp.float32)
    return (jnp.dot(x, wx_ref[:], preferred_element_type=jnp.float32)
            + jnp.dot(oh, wo_ref[:], preferred_element_type=jnp.float32)
            + bcat_ref[:])


def _tc_stage0_body(nt_ref, d_ref, w_ref, b_ref, gns_ref, gnb_ref,
                    wx_ref, wo_ref, bcat_ref, out_ref, *, C, NT):
    x = _down(d_ref[:], w_ref, b_ref, gns_ref, gnb_ref, C)
    out_ref[:] = _table(x, nt_ref, wx_ref, wo_ref, bcat_ref, NT)


def _tc_stage0(nt2, data, w, b, gns, gnb, wx, wo, bcat, *, C, NT, bn):
    N8, CIN = data.shape
    N = N8 // 8
    TW = wx.shape[1]
    return pl.pallas_call(
        functools.partial(_tc_stage0_body, C=C, NT=NT),
        grid=(N // bn,),
        in_specs=[
            pl.BlockSpec((bn, 1), lambda i: (i, 0)),
            pl.BlockSpec((bn * 8, CIN), lambda i: (i, 0)),
            pl.BlockSpec((CIN, C), lambda i: (0, 0)),
            pl.BlockSpec((1, C), lambda i: (0, 0)),
            pl.BlockSpec((1, C), lambda i: (0, 0)),
            pl.BlockSpec((1, C), lambda i: (0, 0)),
            pl.BlockSpec((C, TW), lambda i: (0, 0)),
            pl.BlockSpec((NT, TW), lambda i: (0, 0)),
            pl.BlockSpec((1, TW), lambda i: (0, 0)),
        ],
        out_specs=pl.BlockSpec((bn, TW), lambda i: (i, 0)),
        out_shape=jax.ShapeDtypeStruct((N, TW), jnp.float32),
    )(nt2, data, w, b, gns, gnb, wx, wo, bcat)


def _tc_mid_body(nt_ref, p0_ref, p1_ref, gnsc_ref, gnbc_ref,
                 w_ref, b_ref, gns_ref, gnb_ref, wx_ref, wo_ref, bcat_ref,
                 out_ref, *, C0, C1, NT):
    s = p0_ref[:] + p1_ref[:]
    deg = jnp.maximum(s[:, C0:C0 + 1], 1.0)
    agg = s[:, :C0] / deg
    x1 = jax.nn.gelu(_gn(agg, gnsc_ref[:], gnbc_ref[:], C0))
    x2 = _down(x1, w_ref, b_ref, gns_ref, gnb_ref, C1)
    out_ref[:] = _table(x2, nt_ref, wx_ref, wo_ref, bcat_ref, NT)


def _tc_mid(nt2, p0, p1, gnsc, gnbc, w, b, gns, gnb, wx, wo, bcat,
            *, C0, C1, NT):
    N1, W = p0.shape
    N2 = N1 // 8
    TW = wx.shape[1]
    return pl.pallas_call(
        functools.partial(_tc_mid_body, C0=C0, C1=C1, NT=NT),
        out_shape=jax.ShapeDtypeStruct((N2, TW), jnp.float32),
    )(nt2, p0, p1, gnsc, gnbc, w, b, gns, gnb, wx, wo, bcat)


def _tc_final_body(p0_ref, p1_ref, gns_ref, gnb_ref, out_ref, *, C):
    s = p0_ref[:] + p1_ref[:]
    deg = jnp.maximum(s[:, C:C + 1], 1.0)
    agg = s[:, :C] / deg
    out_ref[:] = jax.nn.gelu(_gn(agg, gns_ref[:], gnb_ref[:], C))


def _tc_final(p0, p1, gns, gnb, *, C):
    N, W = p0.shape
    return pl.pallas_call(
        functools.partial(_tc_final_body, C=C),
        out_shape=jax.ShapeDtypeStruct((N, C), jnp.float32),
    )(p0, p1, gns, gnb)


def _sc_conv(xt2d, ei3, et2, *, NACC, WIDTH, B=128, NBUF=8):
    NTAB, TW = xt2d.shape
    E = ei3.shape[1] * B
    ew = E // NW
    nchunk = ew // B
    nouter = nchunk // NBUF
    rps = NACC // NS
    mesh = plsc.VectorSubcoreMesh(core_axis_name="c", subcore_axis_name="s",
                                  num_cores=NC, num_subcores=NS)

    @functools.partial(
        pl.kernel,
        out_type=jax.ShapeDtypeStruct((NC, NACC, WIDTH), jnp.float32),
        mesh=mesh,
        compiler_params=pltpu.CompilerParams(use_tc_tiling_on_sc=False),
        scratch_types=[
            pltpu.VMEM((nchunk, B), jnp.int32),
            pltpu.VMEM((nchunk, B), jnp.int32),
            pltpu.VMEM((nchunk, B), jnp.int32),
            pltpu.VMEM((nchunk, B), jnp.int32),
            pltpu.VMEM((NBUF, B, WIDTH), jnp.float32),
            pltpu.VMEM_SHARED((NACC, WIDTH), jnp.float32),
            [pltpu.SemaphoreType.DMA] * NBUF,
            [pltpu.SemaphoreType.DMA] * NBUF,
        ],
    )
    def k(xt_hbm, ei_hbm, et_hbm, out_hbm,
          srcb, typb, dstb, idxb, rowsb, acc, gsem, ssem):
        cid = lax.axis_index("c")
        sid = lax.axis_index("s")
        wid = sid * NC + cid
        row0 = wid * nchunk
        xtr = xt_hbm

        pltpu.sync_copy(ei_hbm.at[0, pl.ds(row0, nchunk)], srcb)
        pltpu.sync_copy(ei_hbm.at[1, pl.ds(row0, nchunk)], dstb)
        pltpu.sync_copy(et_hbm.at[pl.ds(row0, nchunk)], typb)

        @pl.loop(0, B)
        def _(i):
            for cc in range(WIDTH // LANES):
                rowsb[0, i, pl.ds(cc * LANES, LANES)] = jnp.zeros(
                    (LANES,), jnp.float32)
        rbase = sid * rps
        for kk in range(rps // B):
            pltpu.sync_copy(rowsb.at[0], acc.at[pl.ds(rbase + kk * B, B)])

        @pl.loop(0, nchunk)
        def _(c):
            for j in range(B // LANES):
                sl = pl.ds(j * LANES, LANES)
                idxb[c, sl] = srcb[c, sl] * NTYPES + typb[c, sl]
        plsc.subcore_barrier()

        def fire_gather(c, b):
            pltpu.async_copy(xtr.at[idxb.at[c]], rowsb.at[b], gsem[b])

        def wait_gather(b):
            pltpu.make_async_copy(
                xtr.at[idxb.at[0]], rowsb.at[b], gsem[b]).wait()

        def fire_scatter(c, b):
            pltpu.async_copy(rowsb.at[b], acc.at[dstb.at[c]], ssem[b],
                             add=True)

        def wait_scatter(b):
            pltpu.make_async_copy(
                rowsb.at[b], acc.at[dstb.at[0]], ssem[b]).wait()

        for b in range(NBUF):
            fire_gather(b, b)

        @pl.loop(0, nouter - 1)
        def _(i):
            c0 = i * NBUF
            for b in range(NBUF):
                wait_gather(b)
                fire_scatter(c0 + b, b)
            for b in range(NBUF):
                wait_scatter(b)
                fire_gather(c0 + NBUF + b, b)

        c0 = (nouter - 1) * NBUF
        for b in range(NBUF):
            wait_gather(b)
            fire_scatter(c0 + b, b)
        for b in range(NBUF):
            wait_scatter(b)

        plsc.subcore_barrier()
        for kk in range(rps // B):
            r0 = rbase + kk * B
            pltpu.sync_copy(acc.at[pl.ds(r0, B)], rowsb.at[0])
            pltpu.sync_copy(rowsb.at[0], out_hbm.at[cid, pl.ds(r0, B)])

    return k(xt2d, ei3, et2)


def _prep_conv_weights(W_conv, C, NT, WPAD):
    Cout = W_conv.shape[2]
    wx = jnp.transpose(W_conv[:, :C, :], (1, 0, 2))
    wx = jnp.pad(wx, ((0, 0), (0, 0), (0, WPAD - Cout)))
    wx = wx.reshape(C, NTYPES * WPAD)
    wo = jnp.transpose(W_conv[:, C:, :], (1, 0, 2))
    wo = jnp.pad(wo, ((0, 0), (0, 0), (0, WPAD - Cout)))
    wo = wo.reshape(NT, NTYPES * WPAD)
    bcat = jnp.zeros((NTYPES, WPAD), jnp.float32).at[:, Cout].set(1.0)
    bcat = bcat.reshape(1, NTYPES * WPAD)
    return wx, wo, bcat


def kernel(data, edge_index_0, edge_type_0, node_type_0,
           edge_index_1, edge_type_1, node_type_1, depth,
           W_down0, b_down0, gns_down0, gnb_down0,
           W_conv0, gns_conv0, gnb_conv0,
           W_down1, b_down1, gns_down1, gnb_down1,
           W_conv1, gns_conv1, gnb_conv1):
    del depth
    N0, C0 = data.shape
    N1 = node_type_0.shape[0]
    N2 = node_type_1.shape[0]
    C1 = W_down1.shape[1]
    NT0 = W_conv0.shape[1] - C0
    NT1 = W_conv1.shape[1] - C1
    W0, W1 = 48, 80
    B = 128

    wx0, wo0, bcat0 = _prep_conv_weights(W_conv0, C0, NT0, W0)
    xt0 = _tc_stage0(node_type_0.reshape(N1, 1), data, W_down0,
                     b_down0.reshape(1, C0), gns_down0.reshape(1, C0),
                     gnb_down0.reshape(1, C0), wx0, wo0, bcat0,
                     C=C0, NT=NT0, bn=2048)
    part0 = _sc_conv(xt0.reshape(N1 * NTYPES, W0), edge_index_0.reshape(2, -1, B),
                     edge_type_0.reshape(-1, B), NACC=N1, WIDTH=W0)

    wx1, wo1, bcat1 = _prep_conv_weights(W_conv1, C1, NT1, W1)
    xt1 = _tc_mid(node_type_1.reshape(N2, 1), part0[0], part0[1],
                  gns_conv0.reshape(1, C0), gnb_conv0.reshape(1, C0),
                  W_down1, b_down1.reshape(1, C1), gns_down1.reshape(1, C1),
                  gnb_down1.reshape(1, C1), wx1, wo1, bcat1,
                  C0=C0, C1=C1, NT=NT1)
    part1 = _sc_conv(xt1.reshape(N2 * NTYPES, W1), edge_index_1.reshape(2, -1, B),
                     edge_type_1.reshape(-1, B), NACC=N2, WIDTH=W1)
    out = _tc_final(part1[0], part1[1], gns_conv1.reshape(1, C1),
                    gnb_conv1.reshape(1, C1), C=C1)
    return out
